# baseline (device time: 203356 ns/iter reference)
import jax
import jax.numpy as jnp
from jax import lax
from jax.experimental import pallas as pl
from jax.experimental.pallas import tpu as pltpu

N_DEV = 16
_GELU_C = 0.7978845608028654


def _gelu(y):
    return 0.5 * y * (1.0 + jnp.tanh(_GELU_C * (y + 0.044715 * y * y * y)))


def kernel(x, w_mat):
    m_per, k = x.shape
    _, n_per = w_mat.shape

    def body(x_ref, w_ref, out_ref, comm_ref, send_sems, recv_sems):
        my = lax.axis_index("i")
        left = lax.rem(my + N_DEV - 1, N_DEV)
        right = lax.rem(my + 1, N_DEV)

        barrier = pltpu.get_barrier_semaphore()
        for nbr in (left, right):
            pl.semaphore_signal(
                barrier, inc=1,
                device_id=(nbr,), device_id_type=pl.DeviceIdType.MESH,
            )
        pl.semaphore_wait(barrier, 2)

        comm_ref[pl.ds(my, 1)] = x_ref[...].reshape(1, m_per, k)

        def slab(origin):
            out_ref[pl.ds(origin * m_per, m_per), :] = _gelu(
                jnp.dot(
                    comm_ref[origin],
                    w_ref[...],
                    preferred_element_type=jnp.float32,
                )
            )

        sends = []
        for h in range(N_DEV - 1):
            o_s = lax.rem(my - h + N_DEV, N_DEV)
            o_r = lax.rem(my - h - 1 + N_DEV, N_DEV)
            send = pltpu.make_async_remote_copy(
                src_ref=comm_ref.at[o_s],
                dst_ref=comm_ref.at[o_s],
                send_sem=send_sems.at[o_s],
                recv_sem=recv_sems.at[o_s],
                device_id=(right,),
                device_id_type=pl.DeviceIdType.MESH,
            )
            send.start()
            sends.append(send)
            recv = pltpu.make_async_remote_copy(
                src_ref=comm_ref.at[o_r],
                dst_ref=comm_ref.at[o_r],
                send_sem=send_sems.at[o_r],
                recv_sem=recv_sems.at[o_r],
                device_id=(left,),
                device_id_type=pl.DeviceIdType.MESH,
            )
            slab(o_s)
            recv.wait_recv()
        slab(lax.rem(my + 1, N_DEV))
        for send in sends:
            send.wait_send()

    out_shape = jax.ShapeDtypeStruct((N_DEV * m_per, n_per), jnp.float32)
    return pl.pallas_call(
        body,
        out_shape=out_shape,
        in_specs=[
            pl.BlockSpec(memory_space=pltpu.VMEM),
            pl.BlockSpec(memory_space=pltpu.VMEM),
        ],
        out_specs=pl.BlockSpec(memory_space=pltpu.VMEM),
        scratch_shapes=[
            pltpu.VMEM((N_DEV, m_per, k), jnp.float32),
            pltpu.SemaphoreType.DMA((N_DEV,)),
            pltpu.SemaphoreType.DMA((N_DEV,)),
        ],
        compiler_params=pltpu.CompilerParams(collective_id=0),
    )(x, w_mat)


# device time: 124274 ns/iter; 1.6364x vs baseline; 1.6364x over previous
import jax
import jax.numpy as jnp
from jax import lax
from jax.experimental import pallas as pl
from jax.experimental.pallas import tpu as pltpu

N_DEV = 16
R_HOPS = N_DEV // 2
L_HOPS = N_DEV - 1 - R_HOPS
_GELU_C = 0.7978845608028654


def _gelu(y):
    return 0.5 * y * (1.0 + jnp.tanh(_GELU_C * (y + 0.044715 * y * y * y)))


def kernel(x, w_mat):
    m_per, k = x.shape
    _, n_per = w_mat.shape

    def body(x_ref, w_ref, out_ref, comm_ref,
             r_send_sems, l_send_sems, recv_sems):
        my = lax.axis_index("i")
        left = lax.rem(my + N_DEV - 1, N_DEV)
        right = lax.rem(my + 1, N_DEV)

        barrier = pltpu.get_barrier_semaphore()
        for nbr in (left, right):
            pl.semaphore_signal(
                barrier, inc=1,
                device_id=(nbr,), device_id_type=pl.DeviceIdType.MESH,
            )
        pl.semaphore_wait(barrier, 2)

        comm_ref[pl.ds(my, 1)] = x_ref[...].reshape(1, m_per, k)

        def slab(origin):
            out_ref[pl.ds(origin * m_per, m_per), :] = _gelu(
                jnp.dot(
                    comm_ref[origin],
                    w_ref[...],
                    preferred_element_type=jnp.float32,
                )
            )

        def copy(slot, send_sem, dev):
            return pltpu.make_async_remote_copy(
                src_ref=comm_ref.at[slot],
                dst_ref=comm_ref.at[slot],
                send_sem=send_sem,
                recv_sem=recv_sems.at[slot],
                device_id=(dev,),
                device_id_type=pl.DeviceIdType.MESH,
            )

        sends = []
        for h in range(R_HOPS):
            o_rf = lax.rem(my - h + N_DEV, N_DEV)
            o_lf = lax.rem(my + h, N_DEV)
            s = copy(o_rf, r_send_sems.at[h], right)
            s.start()
            sends.append(s)
            if h < L_HOPS:
                s = copy(o_lf, l_send_sems.at[h], left)
                s.start()
                sends.append(s)
            if h == 0:
                slab(my)
            else:
                slab(o_rf)
                slab(o_lf)
            copy(lax.rem(my - h - 1 + N_DEV, N_DEV),
                 r_send_sems.at[h], left).wait_recv()
            if h < L_HOPS:
                copy(lax.rem(my + h + 1, N_DEV),
                     l_send_sems.at[h], right).wait_recv()
        slab(lax.rem(my - R_HOPS + N_DEV, N_DEV))
        for s in sends:
            s.wait_send()

    out_shape = jax.ShapeDtypeStruct((N_DEV * m_per, n_per), jnp.float32)
    return pl.pallas_call(
        body,
        out_shape=out_shape,
        in_specs=[
            pl.BlockSpec(memory_space=pltpu.VMEM),
            pl.BlockSpec(memory_space=pltpu.VMEM),
        ],
        out_specs=pl.BlockSpec(memory_space=pltpu.VMEM),
        scratch_shapes=[
            pltpu.VMEM((N_DEV, m_per, k), jnp.float32),
            pltpu.SemaphoreType.DMA((R_HOPS,)),
            pltpu.SemaphoreType.DMA((L_HOPS,)),
            pltpu.SemaphoreType.DMA((N_DEV,)),
        ],
        compiler_params=pltpu.CompilerParams(collective_id=0),
    )(x, w_mat)


# device time: 96398 ns/iter; 2.1095x vs baseline; 1.2892x over previous
import jax
import jax.numpy as jnp
from jax import lax
from jax.experimental import pallas as pl
from jax.experimental.pallas import tpu as pltpu

N_DEV = 16
N_STEPS = N_DEV - 1
_GELU_C = 0.7978845608028654


def _gelu(y):
    return 0.5 * y * (1.0 + jnp.tanh(_GELU_C * (y + 0.044715 * y * y * y)))


def kernel(x, w_mat):
    m_per, k = x.shape
    _, n_per = w_mat.shape
    m_half = m_per // 2

    def body(x_ref, w_ref, out_ref, comm_ref,
             r_send_sems, l_send_sems, recv_sems):
        my = lax.axis_index("i")
        left = lax.rem(my + N_DEV - 1, N_DEV)
        right = lax.rem(my + 1, N_DEV)

        def fr(j):
            return lax.rem(2 * (my - 1 - j // 2) + 2 * N_DEV, 2 * N_DEV) + j % 2

        def gl(j):
            return lax.rem(2 * (my + 1 + j // 2), 2 * N_DEV) + (1 - j % 2)

        barrier = pltpu.get_barrier_semaphore()
        for nbr in (left, right):
            pl.semaphore_signal(
                barrier, inc=1,
                device_id=(nbr,), device_id_type=pl.DeviceIdType.MESH,
            )
        pl.semaphore_wait(barrier, 2)

        comm_ref[pl.ds(2 * my, 2)] = x_ref[...].reshape(2, m_half, k)

        def slab(slot):
            out_ref[pl.ds(slot * m_half, m_half), :] = _gelu(
                jnp.dot(
                    comm_ref[slot],
                    w_ref[...],
                    preferred_element_type=jnp.float32,
                )
            )

        def copy(slot, send_sem, dev):
            return pltpu.make_async_remote_copy(
                src_ref=comm_ref.at[slot],
                dst_ref=comm_ref.at[slot],
                send_sem=send_sem,
                recv_sem=recv_sems.at[slot],
                device_id=(dev,),
                device_id_type=pl.DeviceIdType.MESH,
            )

        sends = []

        def start(slot, sem_arr, j, dev):
            s = copy(slot, sem_arr.at[j], dev)
            s.start()
            sends.append(s)

        start(2 * my, r_send_sems, 0, right)
        start(lax.rem(2 * my + 1, 2 * N_DEV), r_send_sems, 1, right)
        start(lax.rem(2 * my + 1, 2 * N_DEV), l_send_sems, 0, left)
        start(2 * my, l_send_sems, 1, left)
        slab(2 * my)
        slab(lax.rem(2 * my + 1, 2 * N_DEV))

        for j in range(N_STEPS):
            copy(fr(j), r_send_sems.at[0], left).wait_recv()
            copy(gl(j), l_send_sems.at[0], right).wait_recv()
            if j + 2 < N_STEPS:
                start(fr(j), r_send_sems, j + 2, right)
                start(gl(j), l_send_sems, j + 2, left)
            slab(fr(j))
            slab(gl(j))

        for s in sends:
            s.wait_send()

    out_shape = jax.ShapeDtypeStruct((N_DEV * m_per, n_per), jnp.float32)
    return pl.pallas_call(
        body,
        out_shape=out_shape,
        in_specs=[
            pl.BlockSpec(memory_space=pltpu.VMEM),
            pl.BlockSpec(memory_space=pltpu.VMEM),
        ],
        out_specs=pl.BlockSpec(memory_space=pltpu.VMEM),
        scratch_shapes=[
            pltpu.VMEM((2 * N_DEV, m_half, k), jnp.float32),
            pltpu.SemaphoreType.DMA((N_STEPS,)),
            pltpu.SemaphoreType.DMA((N_STEPS,)),
            pltpu.SemaphoreType.DMA((2 * N_DEV,)),
        ],
        compiler_params=pltpu.CompilerParams(collective_id=0),
    )(x, w_mat)


# device time: 76931 ns/iter; 2.6434x vs baseline; 1.2530x over previous
import jax
import jax.numpy as jnp
from jax import lax
from jax.experimental import pallas as pl
from jax.experimental.pallas import tpu as pltpu

N_DEV = 16
_GELU_C = 0.7978845608028654

_ORD = ((0, 1, 2, 3), (1, 2, 0, 3), (2, 3, 1, 0), (3, 2, 1, 0))


def _gelu(y):
    return 0.5 * y * (1.0 + jnp.tanh(_GELU_C * (y + 0.044715 * y * y * y)))


def kernel(x, w_mat):
    m_per, k = x.shape
    _, n_per = w_mat.shape
    m_half = m_per // 2

    def body(x_ref, w_ref, out_ref, comm_ref, recv_sems,
             zup_sems, zdn_sems, pr_col_sems, pl_col_sems,
             pr_diag_sems, pl_diag_sems):
        my = lax.axis_index("i")
        z = my // 4
        q = lax.rem(my, 4)
        base = my - q
        q_l = base + lax.rem(q + 3, 4)
        q_r = base + lax.rem(q + 1, 4)
        up = lax.rem(my + 4, N_DEV)
        dn = lax.rem(my - 4 + N_DEV, N_DEV)
        diag = base + lax.rem(q + 2, 4)

        def col_slot(zp, h):
            return 2 * (4 * zp + q) + h

        barrier = pltpu.get_barrier_semaphore()
        for nbr in (q_l, q_r):
            pl.semaphore_signal(
                barrier, inc=1,
                device_id=(nbr,), device_id_type=pl.DeviceIdType.MESH,
            )

        @pl.when(z < 3)
        def _():
            pl.semaphore_signal(
                barrier, inc=1,
                device_id=(up,), device_id_type=pl.DeviceIdType.MESH,
            )

        @pl.when(z > 0)
        def _():
            pl.semaphore_signal(
                barrier, inc=1,
                device_id=(dn,), device_id_type=pl.DeviceIdType.MESH,
            )

        pl.semaphore_wait(barrier, 2)

        @pl.when(z < 3)
        def _():
            pl.semaphore_wait(barrier, 1)

        @pl.when(z > 0)
        def _():
            pl.semaphore_wait(barrier, 1)

        comm_ref[pl.ds(2 * my, 2)] = x_ref[...].reshape(2, m_half, k)

        def slab(slot):
            out_ref[pl.ds(slot * m_half, m_half), :] = _gelu(
                jnp.dot(
                    comm_ref[slot],
                    w_ref[...],
                    preferred_element_type=jnp.float32,
                )
            )

        def copy(slot, send_sem, dev):
            return pltpu.make_async_remote_copy(
                src_ref=comm_ref.at[slot],
                dst_ref=comm_ref.at[slot],
                send_sem=send_sem,
                recv_sem=recv_sems.at[slot],
                device_id=(dev,),
                device_id_type=pl.DeviceIdType.MESH,
            )

        def wait_recv(slot):
            copy(slot, zup_sems.at[0], q_l).wait_recv()

        for h in (0, 1):
            s = 2 * my + h

            @pl.when(z < 3)
            def _(s=s, h=h):
                copy(s, zup_sems.at[2 * z + h], up).start()

            @pl.when(z > 0)
            def _(s=s, h=h):
                copy(s, zdn_sems.at[2 * z + h], dn).start()

            copy(s, pr_col_sems.at[2 * z + h], q_r).start()
            copy(s, pl_col_sems.at[2 * z + h], q_l).start()
        slab(2 * my)
        slab(2 * my + 1)

        for t in range(3):
            for h in (0, 1):
                zp = z + 1 + t
                s = col_slot(zp, h)
                cond = zp <= 3

                @pl.when(cond)
                def _(s=s, h=h, zp=zp):
                    wait_recv(s)

                @pl.when(cond & (z > 0))
                def _(s=s, h=h, zp=zp):
                    copy(s, zdn_sems.at[2 * zp + h], dn).start()

                @pl.when(cond)
                def _(s=s, h=h, zp=zp):
                    copy(s, pr_col_sems.at[2 * zp + h], q_r).start()
                    copy(s, pl_col_sems.at[2 * zp + h], q_l).start()
                    slab(s)
            for h in (0, 1):
                zp = z - 1 - t
                s = col_slot(zp, h)
                cond = zp >= 0

                @pl.when(cond)
                def _(s=s, h=h, zp=zp):
                    wait_recv(s)

                @pl.when(cond & (z < 3))
                def _(s=s, h=h, zp=zp):
                    copy(s, zup_sems.at[2 * zp + h], up).start()

                @pl.when(cond)
                def _(s=s, h=h, zp=zp):
                    copy(s, pr_col_sems.at[2 * zp + h], q_r).start()
                    copy(s, pl_col_sems.at[2 * zp + h], q_l).start()
                    slab(s)

        def ord_r(r):
            o = jnp.int32(_ORD[3][r])
            for zz in (2, 1, 0):
                o = jnp.where(z == zz, jnp.int32(_ORD[zz][r]), o)
            return o

        for r in range(4):
            o = ord_r(r)
            sl0 = 2 * (4 * o + lax.rem(q + 3, 4))
            sr1 = 2 * (4 * o + lax.rem(q + 1, 4)) + 1
            wait_recv(sl0)
            copy(sl0, pr_diag_sems.at[o], q_r).start()
            slab(sl0)
            wait_recv(sr1)
            copy(sr1, pl_diag_sems.at[o], q_l).start()
            slab(sr1)
            wait_recv(sl0 + 1)
            slab(sl0 + 1)
            wait_recv(sr1 - 1)
            slab(sr1 - 1)
        for r in range(4):
            o = ord_r(r)
            sd = 2 * (4 * o + lax.rem(q + 2, 4))
            wait_recv(sd)
            slab(sd)
            wait_recv(sd + 1)
            slab(sd + 1)

        for zp in range(4):
            for h in (0, 1):
                i = 2 * zp + h
                copy(i, pr_col_sems.at[i], q_r).wait_send()
                copy(i, pl_col_sems.at[i], q_l).wait_send()

                @pl.when((z < 3) & (zp <= z))
                def _(i=i):
                    copy(i, zup_sems.at[i], up).wait_send()

                @pl.when((z > 0) & (zp >= z))
                def _(i=i):
                    copy(i, zdn_sems.at[i], dn).wait_send()

            copy(zp, pr_diag_sems.at[zp], q_r).wait_send()
            copy(zp, pl_diag_sems.at[zp], q_l).wait_send()

    out_shape = jax.ShapeDtypeStruct((N_DEV * m_per, n_per), jnp.float32)
    return pl.pallas_call(
        body,
        out_shape=out_shape,
        in_specs=[
            pl.BlockSpec(memory_space=pltpu.VMEM),
            pl.BlockSpec(memory_space=pltpu.VMEM),
        ],
        out_specs=pl.BlockSpec(memory_space=pltpu.VMEM),
        scratch_shapes=[
            pltpu.VMEM((2 * N_DEV, m_half, k), jnp.float32),
            pltpu.SemaphoreType.DMA((2 * N_DEV,)),
            pltpu.SemaphoreType.DMA((8,)),
            pltpu.SemaphoreType.DMA((8,)),
            pltpu.SemaphoreType.DMA((8,)),
            pltpu.SemaphoreType.DMA((8,)),
            pltpu.SemaphoreType.DMA((4,)),
            pltpu.SemaphoreType.DMA((4,)),
        ],
        compiler_params=pltpu.CompilerParams(collective_id=0),
    )(x, w_mat)
